# baseline (speedup 1.0000x reference)
import functools

import numpy as np
import jax
import jax.numpy as jnp
from jax import lax
from jax.experimental import pallas as pl
from jax.experimental.pallas import tpu as pltpu
from jax.experimental.pallas import tpu_sc as plsc

_NC = 2
_NS = 16
_NW = _NC * _NS
_LANES = 16
_PADE = 128
_BLK = 8


def _pe_table(max_len: int, d_model: int) -> np.ndarray:
    pe = np.zeros((max_len, d_model), dtype=np.float32)
    position = np.arange(max_len, dtype=np.float32)[:, None]
    div_term = np.exp(
        np.arange(0, d_model, 2, dtype=np.float32) * -(np.log(10000.0) / d_model)
    )
    pe[:, 0::2] = np.sin(position * div_term)
    pe[:, 1::2] = np.cos(position * div_term)
    return pe


@functools.lru_cache(maxsize=None)
def _build(B: int, S: int, V: int, E: int):
    assert B % (_NW * _PADE) == 0 and E % _LANES == 0 and S % _BLK == 0
    bw = B // _NW
    assert bw == _PADE

    mesh = plsc.VectorSubcoreMesh(
        core_axis_name="c", subcore_axis_name="s",
        num_cores=_NC, num_subcores=_NS,
    )

    @functools.partial(
        pl.kernel,
        out_type=jax.ShapeDtypeStruct((S, E, B), jnp.float32),
        mesh=mesh,
        compiler_params=pltpu.CompilerParams(
            use_tc_tiling_on_sc=True, needs_layout_passes=False
        ),
        scratch_types=[
            pltpu.VMEM((2 * _BLK, _PADE), jnp.int32),
            [pltpu.VMEM((bw, _PADE), jnp.float32) for _ in range(2)],
            [pltpu.VMEM((E, _PADE), jnp.float32) for _ in range(2)],
            pltpu.VMEM((S * E,), jnp.float32),
            [pltpu.SemaphoreType.DMA for _ in range(4)],
        ],
    )
    def emb(seq_hbm, table_hbm, pe_hbm, out_hbm, idx_v, rbufs, obufs, pe_v, sems):
        sg, so = sems[:2], sems[2:]
        wid = lax.axis_index("s") * _NC + lax.axis_index("c")
        col0 = pl.multiple_of(wid * bw, _PADE)
        pltpu.sync_copy(pe_hbm, pe_v)

        def stage_idx_block(s_first):
            if not isinstance(s_first, int):
                s_first = pl.multiple_of(s_first, _BLK)
            pltpu.sync_copy(
                seq_hbm.at[pl.ds(s_first, _BLK), pl.ds(col0, _PADE)],
                idx_v.at[pl.ds(s_first % (2 * _BLK), _BLK)],
            )

        def fire_gather(s, b):
            pltpu.async_copy(
                table_hbm.at[idx_v.at[s % (2 * _BLK)]], rbufs[b], sg[b]
            )

        def wait_gather(s, b):
            pltpu.make_async_copy(
                table_hbm.at[idx_v.at[s % (2 * _BLK)]], rbufs[b], sg[b]
            ).wait()

        def fire_out(s, b):
            pltpu.async_copy(
                obufs[b], out_hbm.at[s, :, pl.ds(col0, _PADE)], so[b]
            )

        def wait_out(s_prev, b):
            pltpu.make_async_copy(
                obufs[b], out_hbm.at[s_prev, :, pl.ds(col0, _PADE)], so[b]
            ).wait()

        def transpose_add(s, b):
            rb = rbufs[b]
            ob = obufs[b]
            lane = lax.iota(jnp.int32, _LANES)

            def gbody(g, c):
                pe_chunk = pe_v[pl.ds(s * E + g * _LANES, _LANES)]
                rows = [lane + (k * _LANES) for k in range(bw // _LANES)]
                for e16 in range(_LANES):
                    e = g * _LANES + e16
                    pe_s = pe_chunk[e16]
                    cols = jnp.full((_LANES,), e, jnp.int32)
                    for k in range(bw // _LANES):
                        v = plsc.load_gather(rb, [rows[k], cols]) + pe_s
                        ob[e, pl.ds(k * _LANES, _LANES)] = v
                return c

            lax.fori_loop(0, E // _LANES, gbody, 0)

        def step(s, b):
            wait_gather(s, b)

            @pl.when(s >= 2)
            def _():
                wait_out(s - 2, b)

            transpose_add(s, b)
            fire_out(s, b)

            @pl.when(s + 2 < S)
            def _():
                @pl.when((s + 2) % _BLK == 0)
                def _():
                    stage_idx_block(s + 2)

                fire_gather(s + 2, b)

        stage_idx_block(0)
        fire_gather(0, 0)
        fire_gather(1, 1)

        def outer_body(i, carry):
            s0 = 2 * i
            step(s0, 0)
            step(s0 + 1, 1)
            return carry

        lax.fori_loop(0, S // 2, outer_body, 0)

        wait_out(S - 2, 0)
        wait_out(S - 1, 1)

    pe_host = jnp.asarray(_pe_table(S, E).reshape(-1))

    def run(sequence, token_table):
        seq_t = jnp.transpose(sequence)
        table_p = jnp.pad(token_table, ((0, 0), (0, _PADE - E)))
        out_seb = emb(seq_t, table_p, pe_host)
        return jnp.transpose(out_seb, (2, 0, 1))

    return run


def kernel(sequence, token_table):
    B, S = sequence.shape
    V, E = token_table.shape
    return _build(B, S, V, E)(sequence, token_table)


# speedup vs baseline: 1.9250x; 1.9250x over previous
import functools

import numpy as np
import jax
import jax.numpy as jnp
from jax import lax
from jax.experimental import pallas as pl
from jax.experimental.pallas import tpu as pltpu
from jax.experimental.pallas import tpu_sc as plsc

_NC = 2
_NS = 16
_NW = _NC * _NS
_LANES = 16
_PADE = 128
_BLK = 8


def _pe_table(max_len: int, d_model: int) -> np.ndarray:
    pe = np.zeros((max_len, d_model), dtype=np.float32)
    position = np.arange(max_len, dtype=np.float32)[:, None]
    div_term = np.exp(
        np.arange(0, d_model, 2, dtype=np.float32) * -(np.log(10000.0) / d_model)
    )
    pe[:, 0::2] = np.sin(position * div_term)
    pe[:, 1::2] = np.cos(position * div_term)
    return pe


@functools.lru_cache(maxsize=None)
def _build(B: int, S: int, V: int, E: int):
    assert B % (_NW * _PADE) == 0 and E % _LANES == 0 and S % _BLK == 0
    bw = B // _NW
    assert bw == _PADE

    mesh = plsc.VectorSubcoreMesh(
        core_axis_name="c", subcore_axis_name="s",
        num_cores=_NC, num_subcores=_NS,
    )

    @functools.partial(
        pl.kernel,
        out_type=jax.ShapeDtypeStruct((S, E, B), jnp.float32),
        mesh=mesh,
        compiler_params=pltpu.CompilerParams(
            use_tc_tiling_on_sc=True, needs_layout_passes=False
        ),
        scratch_types=[
            pltpu.VMEM((2 * _BLK, _PADE), jnp.int32),
            [pltpu.VMEM((bw, _PADE), jnp.float32) for _ in range(2)],
            [pltpu.VMEM((E, _PADE), jnp.float32) for _ in range(2)],
            pltpu.VMEM((S * E,), jnp.float32),
            [pltpu.SemaphoreType.DMA for _ in range(4)],
        ],
    )
    def emb(seq_hbm, table_hbm, pe_hbm, out_hbm, idx_v, rbufs, obufs, pe_v, sems):
        sg, so = sems[:2], sems[2:]
        wid = lax.axis_index("s") * _NC + lax.axis_index("c")
        col0 = pl.multiple_of(wid * bw, _PADE)
        pltpu.sync_copy(pe_hbm, pe_v)

        def stage_idx_block(s_first):
            if not isinstance(s_first, int):
                s_first = pl.multiple_of(s_first, _BLK)
            pltpu.sync_copy(
                seq_hbm.at[pl.ds(s_first, _BLK), pl.ds(col0, _PADE)],
                idx_v.at[pl.ds(s_first % (2 * _BLK), _BLK)],
            )

        def fire_gather(s, b):
            pltpu.async_copy(
                table_hbm.at[idx_v.at[s % (2 * _BLK)]], rbufs[b], sg[b]
            )

        def wait_gather(s, b):
            pltpu.make_async_copy(
                table_hbm.at[idx_v.at[s % (2 * _BLK)]], rbufs[b], sg[b]
            ).wait()

        def fire_out(s, b):
            pltpu.async_copy(
                obufs[b], out_hbm.at[s, :, pl.ds(col0, _PADE)], so[b]
            )

        def wait_out(s_prev, b):
            pltpu.make_async_copy(
                obufs[b], out_hbm.at[s_prev, :, pl.ds(col0, _PADE)], so[b]
            ).wait()

        def transpose_add(s, b):
            rb = rbufs[b]
            ob = obufs[b]
            lane = lax.iota(jnp.int32, _LANES)
            rows = [lane + (k * _LANES) for k in range(bw // _LANES)]

            @plsc.parallel_loop(0, E, 1, unroll=8)
            def ebody(e):
                pe_bc = plsc.load_gather(
                    pe_v, [jnp.full((_LANES,), s * E + e, jnp.int32)]
                )
                cols = jnp.full((_LANES,), e, jnp.int32)
                for k in range(bw // _LANES):
                    v = plsc.load_gather(rb, [rows[k], cols]) + pe_bc
                    ob[e, pl.ds(k * _LANES, _LANES)] = v

        def step(s, b):
            wait_gather(s, b)

            @pl.when(s >= 2)
            def _():
                wait_out(s - 2, b)

            transpose_add(s, b)
            fire_out(s, b)

            @pl.when(s + 2 < S)
            def _():
                @pl.when((s + 2) % _BLK == 0)
                def _():
                    stage_idx_block(s + 2)

                fire_gather(s + 2, b)

        stage_idx_block(0)
        fire_gather(0, 0)
        fire_gather(1, 1)

        def outer_body(i, carry):
            s0 = 2 * i
            step(s0, 0)
            step(s0 + 1, 1)
            return carry

        lax.fori_loop(0, S // 2, outer_body, 0)

        wait_out(S - 2, 0)
        wait_out(S - 1, 1)

    pe_host = jnp.asarray(_pe_table(S, E).reshape(-1))

    def run(sequence, token_table):
        seq_t = jnp.transpose(sequence)
        table_p = jnp.pad(token_table, ((0, 0), (0, _PADE - E)))
        out_seb = emb(seq_t, table_p, pe_host)
        return jnp.transpose(out_seb, (2, 0, 1))

    return run


def kernel(sequence, token_table):
    B, S = sequence.shape
    V, E = token_table.shape
    return _build(B, S, V, E)(sequence, token_table)


# speedup vs baseline: 2.3053x; 1.1975x over previous
import functools

import numpy as np
import jax
import jax.numpy as jnp
from jax import lax
from jax.experimental import pallas as pl
from jax.experimental.pallas import tpu as pltpu
from jax.experimental.pallas import tpu_sc as plsc

_NC = 2
_NS = 16
_NW = _NC * _NS
_LANES = 16
_PADE = 128
_BLK = 8


def _pe_table(max_len: int, d_model: int) -> np.ndarray:
    pe = np.zeros((max_len, d_model), dtype=np.float32)
    position = np.arange(max_len, dtype=np.float32)[:, None]
    div_term = np.exp(
        np.arange(0, d_model, 2, dtype=np.float32) * -(np.log(10000.0) / d_model)
    )
    pe[:, 0::2] = np.sin(position * div_term)
    pe[:, 1::2] = np.cos(position * div_term)
    return pe


@functools.lru_cache(maxsize=None)
def _build(B: int, S: int, V: int, E: int):
    assert B % (_NW * _PADE) == 0 and E % _LANES == 0 and S % _BLK == 0
    bw = B // _NW
    assert bw == _PADE

    mesh = plsc.VectorSubcoreMesh(
        core_axis_name="c", subcore_axis_name="s",
        num_cores=_NC, num_subcores=_NS,
    )

    @functools.partial(
        pl.kernel,
        out_type=jax.ShapeDtypeStruct((S, E, B), jnp.float32),
        mesh=mesh,
        compiler_params=pltpu.CompilerParams(
            use_tc_tiling_on_sc=True, needs_layout_passes=False
        ),
        scratch_types=[
            pltpu.VMEM((2 * _BLK, _PADE), jnp.int32),
            [pltpu.VMEM((bw, _PADE), jnp.float32) for _ in range(2)],
            [pltpu.VMEM((E, _PADE), jnp.float32) for _ in range(2)],
            pltpu.VMEM((S * E,), jnp.float32),
            pltpu.VMEM(((bw // _LANES) * (E // _LANES) * _LANES * 17,), jnp.float32),
            [pltpu.SemaphoreType.DMA for _ in range(4)],
        ],
    )
    def emb(seq_hbm, table_hbm, pe_hbm, out_hbm, idx_v, rbufs, obufs, pe_v, skew, sems):
        sg, so = sems[:2], sems[2:]
        wid = lax.axis_index("s") * _NC + lax.axis_index("c")
        col0 = pl.multiple_of(wid * bw, _PADE)
        pltpu.sync_copy(pe_hbm, pe_v)

        def stage_idx_block(s_first):
            if not isinstance(s_first, int):
                s_first = pl.multiple_of(s_first, _BLK)
            pltpu.sync_copy(
                seq_hbm.at[pl.ds(s_first, _BLK), pl.ds(col0, _PADE)],
                idx_v.at[pl.ds(s_first % (2 * _BLK), _BLK)],
            )

        def fire_gather(s, b):
            pltpu.async_copy(
                table_hbm.at[idx_v.at[s % (2 * _BLK)]], rbufs[b], sg[b]
            )

        def wait_gather(s, b):
            pltpu.make_async_copy(
                table_hbm.at[idx_v.at[s % (2 * _BLK)]], rbufs[b], sg[b]
            ).wait()

        def fire_out(s, b):
            pltpu.async_copy(
                obufs[b], out_hbm.at[s, :, pl.ds(col0, _PADE)], so[b]
            )

        def wait_out(s_prev, b):
            pltpu.make_async_copy(
                obufs[b], out_hbm.at[s_prev, :, pl.ds(col0, _PADE)], so[b]
            ).wait()

        def transpose_add(s, b):
            rb = rbufs[b]
            ob = obufs[b]
            lane = lax.iota(jnp.int32, _LANES)
            w_base = lane * 17

            blk_sz = _LANES * 17

            @plsc.parallel_loop(0, bw // _LANES, 1, unroll=2)
            def kbody(k):
                kb = k * ((E // _LANES) * blk_sz)
                for g in range(E // _LANES):
                    base = kb + g * blk_sz
                    pe_g = pe_v[pl.ds(s * E + g * _LANES, _LANES)]
                    for j in range(_LANES):
                        v = rb[k * _LANES + j, pl.ds(g * _LANES, _LANES)] + pe_g
                        plsc.store_scatter(skew, [w_base + (base + j)], v)
                    for l in range(_LANES):
                        rd = plsc.load_gather(skew, [lane + (base + l * 17)])
                        ob[g * _LANES + l, pl.ds(k * _LANES, _LANES)] = rd

        def step(s, b):
            wait_gather(s, b)

            @pl.when(s >= 2)
            def _():
                wait_out(s - 2, b)

            transpose_add(s, b)
            fire_out(s, b)

            @pl.when(s + 2 < S)
            def _():
                @pl.when((s + 2) % _BLK == 0)
                def _():
                    stage_idx_block(s + 2)

                fire_gather(s + 2, b)

        stage_idx_block(0)
        fire_gather(0, 0)
        fire_gather(1, 1)

        def outer_body(i, carry):
            s0 = 2 * i
            step(s0, 0)
            step(s0 + 1, 1)
            return carry

        lax.fori_loop(0, S // 2, outer_body, 0)

        wait_out(S - 2, 0)
        wait_out(S - 1, 1)

    pe_host = jnp.asarray(_pe_table(S, E).reshape(-1))

    def run(sequence, token_table):
        seq_t = jnp.transpose(sequence)
        table_p = jnp.pad(token_table, ((0, 0), (0, _PADE - E)))
        out_seb = emb(seq_t, table_p, pe_host)
        return jnp.transpose(out_seb, (2, 0, 1))

    return run


def kernel(sequence, token_table):
    B, S = sequence.shape
    V, E = token_table.shape
    return _build(B, S, V, E)(sequence, token_table)
